# Initial kernel scaffold; baseline (speedup 1.0000x reference)
import jax
import jax.numpy as jnp
from jax.experimental import pallas as pl


def kernel(embedding_sequence, w_ih, w_hh, b_ih, b_hh):
    raise NotImplementedError("write your pallas kernel here")


# speedup vs baseline: 8.1934x; 8.1934x over previous
import jax
import jax.numpy as jnp
from jax.experimental import pallas as pl
from jax.experimental.pallas import tpu as pltpu

B, S, IN_DIM, HID = 128, 512, 512, 1024
TB = 8
NCORES = 2
BC = B // NCORES


def _gru_kernel(x_ref, wih_ref, whh_ref, bih_ref, bhh_ref, out_ref,
                h_ref, gi_ref):
    t = pl.program_id(1)

    @pl.when(t == 0)
    def _():
        h_ref[...] = jnp.zeros_like(h_ref)

    x = x_ref[...].reshape(TB * BC, IN_DIM)
    gi_ref[...] = (
        jnp.dot(x, wih_ref[...], preferred_element_type=jnp.float32)
        + bih_ref[...]
    )

    h = h_ref[...]
    whh = whh_ref[...]
    bhh = bhh_ref[...]
    for i in range(TB):
        gh = jnp.dot(h, whh, preferred_element_type=jnp.float32) + bhh
        gi = gi_ref[i * BC:(i + 1) * BC, :]
        r = jax.nn.sigmoid(gi[:, :HID] + gh[:, :HID])
        z = jax.nn.sigmoid(gi[:, HID:2 * HID] + gh[:, HID:2 * HID])
        n = jnp.tanh(gi[:, 2 * HID:] + r * gh[:, 2 * HID:])
        h = (1.0 - z) * n + z * h
        out_ref[i] = h
    h_ref[...] = h


def _gru_pallas(x_sbi, w_ih_t, w_hh_t, b_ih, b_hh, *, interpret=False):
    return pl.pallas_call(
        _gru_kernel,
        out_shape=jax.ShapeDtypeStruct((S, B, HID), jnp.float32),
        grid=(NCORES, S // TB),
        in_specs=[
            pl.BlockSpec((TB, BC, IN_DIM), lambda c, t: (t, c, 0)),
            pl.BlockSpec(memory_space=pltpu.VMEM),
            pl.BlockSpec(memory_space=pltpu.VMEM),
            pl.BlockSpec(memory_space=pltpu.VMEM),
            pl.BlockSpec(memory_space=pltpu.VMEM),
        ],
        out_specs=pl.BlockSpec((TB, BC, HID), lambda c, t: (t, c, 0)),
        scratch_shapes=[
            pltpu.VMEM((BC, HID), jnp.float32),
            pltpu.VMEM((TB * BC, 3 * HID), jnp.float32),
        ],
        compiler_params=pltpu.CompilerParams(
            dimension_semantics=("parallel", "arbitrary"),
            vmem_limit_bytes=56 * 1024 * 1024,
        ),
        name="gru_fused",
        interpret=interpret,
    )(x_sbi, w_ih_t, w_hh_t, b_ih, b_hh)


def kernel(embedding_sequence, w_ih, w_hh, b_ih, b_hh, *, interpret=False):
    x_sbi = jnp.swapaxes(embedding_sequence, 0, 1)
    hiddens_sbh = _gru_pallas(
        x_sbi,
        w_ih.T,
        w_hh.T,
        b_ih.reshape(1, -1),
        b_hh.reshape(1, -1),
        interpret=interpret,
    )
    return jnp.swapaxes(hiddens_sbh, 0, 1), hiddens_sbh[-1]


# speedup vs baseline: 13.6264x; 1.6631x over previous
import jax
import jax.numpy as jnp
from jax.experimental import pallas as pl
from jax.experimental.pallas import tpu as pltpu

B, S, IN_DIM, HID = 128, 512, 512, 1024
TB = 8
BC = B


def _gru_kernel(x_ref, wih_ref, whh_ref, bih_ref, bhh_ref, out_ref,
                h_ref, gi_ref):
    t = pl.program_id(0)

    @pl.when(t == 0)
    def _():
        h_ref[...] = jnp.zeros_like(h_ref)

    x = x_ref[...].reshape(TB * BC, IN_DIM)
    gi_ref[...] = (
        jnp.dot(x, wih_ref[...], preferred_element_type=jnp.float32)
        + bih_ref[...]
    )

    h = h_ref[...]
    whh = whh_ref[...]
    bhh = bhh_ref[...]
    for i in range(TB):
        gh = jnp.dot(h, whh, preferred_element_type=jnp.float32) + bhh
        gi = gi_ref[i * BC:(i + 1) * BC, :]
        r = jax.nn.sigmoid(gi[:, :HID] + gh[:, :HID])
        z = jax.nn.sigmoid(gi[:, HID:2 * HID] + gh[:, HID:2 * HID])
        n = jnp.tanh(gi[:, 2 * HID:] + r * gh[:, 2 * HID:])
        h = (1.0 - z) * n + z * h
        out_ref[i] = h
    h_ref[...] = h


def _gru_pallas(x_sbi, w_ih_t, w_hh_t, b_ih, b_hh, *, interpret=False):
    return pl.pallas_call(
        _gru_kernel,
        out_shape=jax.ShapeDtypeStruct((S, B, HID), jnp.float32),
        grid=(S // TB,),
        in_specs=[
            pl.BlockSpec((TB, BC, IN_DIM), lambda t: (t, 0, 0)),
            pl.BlockSpec(memory_space=pltpu.VMEM),
            pl.BlockSpec(memory_space=pltpu.VMEM),
            pl.BlockSpec(memory_space=pltpu.VMEM),
            pl.BlockSpec(memory_space=pltpu.VMEM),
        ],
        out_specs=pl.BlockSpec((TB, BC, HID), lambda t: (t, 0, 0)),
        scratch_shapes=[
            pltpu.VMEM((BC, HID), jnp.float32),
            pltpu.VMEM((TB * BC, 3 * HID), jnp.float32),
        ],
        compiler_params=pltpu.CompilerParams(
            dimension_semantics=("arbitrary",),
            vmem_limit_bytes=56 * 1024 * 1024,
        ),
        name="gru_fused",
        interpret=interpret,
    )(x_sbi, w_ih_t, w_hh_t, b_ih, b_hh)


def kernel(embedding_sequence, w_ih, w_hh, b_ih, b_hh, *, interpret=False):
    x_sbi = jnp.swapaxes(embedding_sequence, 0, 1)
    hiddens_sbh = _gru_pallas(
        x_sbi,
        w_ih.T,
        w_hh.T,
        b_ih.reshape(1, -1),
        b_hh.reshape(1, -1),
        interpret=interpret,
    )
    return jnp.swapaxes(hiddens_sbh, 0, 1), hiddens_sbh[-1]


# speedup vs baseline: 16.3386x; 1.1990x over previous
import jax
import jax.numpy as jnp
from jax.experimental import pallas as pl
from jax.experimental.pallas import tpu as pltpu

B, S, IN_DIM, HID = 128, 512, 512, 1024
TB = 8
BC = B


def _gru_kernel(x_ref, wih_ref, whh_ref, bih_ref, bhh_ref, out_ref, h_ref):
    t = pl.program_id(0)

    @pl.when(t == 0)
    def _():
        h_ref[...] = jnp.zeros_like(h_ref)

    h = h_ref[...]
    whh = whh_ref[...]
    wih = wih_ref[...]
    bih = bih_ref[...]
    bhh = bhh_ref[...]
    for i in range(TB):
        xi = x_ref[:, i, :].reshape(BC, IN_DIM)
        gi = jnp.dot(xi, wih, preferred_element_type=jnp.float32) + bih
        gh = jnp.dot(h, whh, preferred_element_type=jnp.float32) + bhh
        r = jax.nn.sigmoid(gi[:, :HID] + gh[:, :HID])
        z = jax.nn.sigmoid(gi[:, HID:2 * HID] + gh[:, HID:2 * HID])
        n = jnp.tanh(gi[:, 2 * HID:] + r * gh[:, 2 * HID:])
        h = (1.0 - z) * n + z * h
        out_ref[:, i, :] = h
    h_ref[...] = h


def _gru_pallas(x_bsi, w_ih_t, w_hh_t, b_ih, b_hh, *, interpret=False):
    return pl.pallas_call(
        _gru_kernel,
        out_shape=jax.ShapeDtypeStruct((B, S, HID), jnp.float32),
        grid=(S // TB,),
        in_specs=[
            pl.BlockSpec((BC, TB, IN_DIM), lambda t: (0, t, 0)),
            pl.BlockSpec(memory_space=pltpu.VMEM),
            pl.BlockSpec(memory_space=pltpu.VMEM),
            pl.BlockSpec(memory_space=pltpu.VMEM),
            pl.BlockSpec(memory_space=pltpu.VMEM),
        ],
        out_specs=pl.BlockSpec((BC, TB, HID), lambda t: (0, t, 0)),
        scratch_shapes=[
            pltpu.VMEM((BC, HID), jnp.float32),
        ],
        compiler_params=pltpu.CompilerParams(
            dimension_semantics=("arbitrary",),
            vmem_limit_bytes=56 * 1024 * 1024,
        ),
        name="gru_fused",
        interpret=interpret,
    )(x_bsi, w_ih_t, w_hh_t, b_ih, b_hh)


def kernel(embedding_sequence, w_ih, w_hh, b_ih, b_hh, *, interpret=False):
    hiddens = _gru_pallas(
        embedding_sequence,
        w_ih.T,
        w_hh.T,
        b_ih.reshape(1, -1),
        b_hh.reshape(1, -1),
        interpret=interpret,
    )
    return hiddens, hiddens[:, -1]


# speedup vs baseline: 17.0888x; 1.0459x over previous
import jax
import jax.numpy as jnp
from jax.experimental import pallas as pl
from jax.experimental.pallas import tpu as pltpu

B, S, IN_DIM, HID = 128, 512, 512, 1024
TB = 8
QG = 1
BC = B


def _mdot(a, b):
    return jax.lax.dot_general(
        a, b, (((1,), (0,)), ((), ())), preferred_element_type=jnp.float32)


def _gru_kernel(x_ref, wih_ref, whh_ref, b1_ref, bhn_ref, out_ref,
                hlast_ref, h_ref):
    t = pl.program_id(0)

    @pl.when(t == 0)
    def _():
        h_ref[...] = jnp.zeros_like(h_ref)

    h = h_ref[...]
    whh = whh_ref[...]
    wih = wih_ref[...]
    b1 = b1_ref[...]
    bhn = bhn_ref[...]
    for q in range(TB // QG):
        xq = jnp.concatenate(
            [x_ref[:, q * QG + j, :].reshape(BC, IN_DIM) for j in range(QG)],
            axis=0,
        )
        giq = _mdot(xq, wih) + b1
        for j in range(QG):
            gi = giq[j * BC:(j + 1) * BC, :]
            gh = _mdot(h, whh)
            r = 0.5 + 0.5 * jnp.tanh(0.5 * (gi[:, :HID] + gh[:, :HID]))
            z = 0.5 + 0.5 * jnp.tanh(
                0.5 * (gi[:, HID:2 * HID] + gh[:, HID:2 * HID]))
            n = jnp.tanh(gi[:, 2 * HID:] + r * (gh[:, 2 * HID:] + bhn))
            h = n + z * (h - n)
            out_ref[:, q * QG + j, :] = h
    h_ref[...] = h
    hlast_ref[...] = h


def _gru_pallas(x_bsi, w_ih_t, w_hh_t, b_ih, b_hh, *, interpret=False):
    return pl.pallas_call(
        _gru_kernel,
        out_shape=(
            jax.ShapeDtypeStruct((B, S, HID), jnp.float32),
            jax.ShapeDtypeStruct((B, HID), jnp.float32),
        ),
        grid=(S // TB,),
        in_specs=[
            pl.BlockSpec((BC, TB, IN_DIM), lambda t: (0, t, 0)),
            pl.BlockSpec(memory_space=pltpu.VMEM),
            pl.BlockSpec(memory_space=pltpu.VMEM),
            pl.BlockSpec(memory_space=pltpu.VMEM),
            pl.BlockSpec(memory_space=pltpu.VMEM),
        ],
        out_specs=(
            pl.BlockSpec((BC, TB, HID), lambda t: (0, t, 0)),
            pl.BlockSpec((BC, HID), lambda t: (0, 0)),
        ),
        scratch_shapes=[
            pltpu.VMEM((BC, HID), jnp.float32),
        ],
        compiler_params=pltpu.CompilerParams(
            dimension_semantics=("arbitrary",),
            vmem_limit_bytes=56 * 1024 * 1024,
        ),
        name="gru_fused",
        interpret=interpret,
    )(x_bsi, w_ih_t, w_hh_t, b_ih, b_hh)


def kernel(embedding_sequence, w_ih, w_hh, b_ih, b_hh, *, interpret=False):
    b1 = jnp.concatenate(
        [(b_ih[:2 * HID] + b_hh[:2 * HID]), b_ih[2 * HID:]]
    ).reshape(1, 3 * HID)
    bhn = b_hh[2 * HID:].reshape(1, HID)
    hiddens, h_last = _gru_pallas(
        embedding_sequence,
        w_ih.T.astype(jnp.bfloat16),
        w_hh.T.astype(jnp.bfloat16),
        b1,
        bhn,
        interpret=interpret,
    )
    return hiddens, h_last
